# Initial kernel scaffold; baseline (speedup 1.0000x reference)
import jax
import jax.numpy as jnp
from jax.experimental import pallas as pl


def kernel(cls0, reg0, cls1, reg1, cls2, reg2):
    raise NotImplementedError("write your pallas kernel here")


# speedup vs baseline: 2.2675x; 2.2675x over previous
import functools

import jax
import jax.numpy as jnp
import numpy as np
from jax.experimental import pallas as pl
from jax.experimental.pallas import tpu as pltpu

_NUM_CLASSES = 80
_NUM_ANCHORS = 3
_STRIDES = (8, 16, 32)
_TOPK = 1000
_CONF_THRESH = 0.05
_NMS_THRESH = 0.6
_ANCHOR_SIZE = np.array(
    [[10, 13], [16, 30], [33, 23], [30, 61], [62, 45], [59, 119],
     [116, 90], [156, 198], [373, 326]], dtype=np.float32).reshape(3, 3, 2)

_N = 3 * _TOPK
_P = 3072
_ROWS = _P // 128


def _nms_kernel(x1s, y1s, x2s, y2s, ars,
                x1v, y1v, x2v, y2v, arv,
                sc_v, lb_v,
                keep_out, sc_out, lb_out):
    idx = (jax.lax.broadcasted_iota(jnp.int32, (_ROWS, 128), 0) * 128
           + jax.lax.broadcasted_iota(jnp.int32, (_ROWS, 128), 1))
    scores = sc_v[...]
    alive0 = jnp.where(scores > _CONF_THRESH, 1.0, 0.0).astype(jnp.float32)

    x1 = x1v[...]
    y1 = y1v[...]
    x2 = x2v[...]
    y2 = y2v[...]
    ar = arv[...]

    def body(i, alive):
        onehot = (idx == i)
        ai = jnp.sum(jnp.where(onehot, alive, 0.0), axis=(0, 1), keepdims=True)
        x1i = x1s[i]
        y1i = y1s[i]
        x2i = x2s[i]
        y2i = y2s[i]
        ari = ars[i]
        xx1 = jnp.maximum(x1, x1i)
        yy1 = jnp.maximum(y1, y1i)
        xx2 = jnp.minimum(x2, x2i)
        yy2 = jnp.minimum(y2, y2i)
        inter = (jnp.maximum(xx2 - xx1, 0.0) * jnp.maximum(yy2 - yy1, 0.0))
        denom = ar + ari - inter + 1e-10
        sup = jnp.where((inter > _NMS_THRESH * denom) & (idx > i), 1.0, 0.0)
        return alive * (1.0 - ai * sup)

    alive = jax.lax.fori_loop(0, _N, body, alive0)
    keep_out[...] = alive
    sc_out[...] = scores * alive
    lb_out[...] = jnp.where(alive > 0.5, lb_v[...], -1)


def _run_nms(boxes, scores, labels):
    pad = _P - _N
    mx = jnp.max(boxes)
    off = labels.astype(jnp.float32) * (mx + 1.0)
    x1 = boxes[:, 0] + off
    y1 = boxes[:, 1] + off
    x2 = boxes[:, 2] + off
    y2 = boxes[:, 3] + off
    ar = (x2 - x1) * (y2 - y1)

    def padv(v, val):
        return jnp.pad(v, (0, pad), constant_values=val)

    x1p = padv(x1, 0.0)
    y1p = padv(y1, 0.0)
    x2p = padv(x2, 0.0)
    y2p = padv(y2, 0.0)
    arp = padv(ar, 0.0)
    scp = padv(scores, 0.0)
    lbp = padv(labels, 0)

    smem = pl.BlockSpec(memory_space=pltpu.SMEM)
    r2 = lambda v: v.reshape(_ROWS, 128)
    out = pl.pallas_call(
        _nms_kernel,
        in_specs=[smem] * 5 + [pl.BlockSpec(memory_space=pltpu.VMEM)] * 7,
        out_specs=[pl.BlockSpec(memory_space=pltpu.VMEM)] * 3,
        out_shape=[
            jax.ShapeDtypeStruct((_ROWS, 128), jnp.float32),
            jax.ShapeDtypeStruct((_ROWS, 128), jnp.float32),
            jax.ShapeDtypeStruct((_ROWS, 128), jnp.int32),
        ],
    )(x1p, y1p, x2p, y2p, arp,
      r2(x1p), r2(y1p), r2(x2p), r2(y2p), r2(arp),
      r2(scp), r2(lbp))
    keep, sc, lb = (o.reshape(_P)[:_N] for o in out)
    return keep, sc, lb


def _decode_level(cls_p, reg_p, stride, asize):
    H, W = cls_p.shape[2], cls_p.shape[3]
    cls = cls_p[0].transpose(1, 2, 0).reshape(-1)
    scores_all = jax.nn.sigmoid(cls)
    top_v, top_i = jax.lax.top_k(scores_all, _TOPK)
    anchor_idx = top_i // _NUM_CLASSES
    labels = top_i % _NUM_CLASSES
    reg = reg_p[0].transpose(1, 2, 0).reshape(-1, 4)
    regs = reg[anchor_idx]
    a = anchor_idx % _NUM_ANCHORS
    cell = anchor_idx // _NUM_ANCHORS
    gx = (cell % W).astype(jnp.float32)
    gy = (cell // W).astype(jnp.float32)
    asz = jnp.asarray(asize, jnp.float32)
    aw = asz[a, 0]
    ah = asz[a, 1]
    cx = (gx + 0.5) * stride + regs[:, 0] * stride
    cy = (gy + 0.5) * stride + regs[:, 1] * stride
    w = jnp.exp(regs[:, 2]) * aw
    h = jnp.exp(regs[:, 3]) * ah
    boxes = jnp.stack([cx - 0.5 * w, cy - 0.5 * h,
                       cx + 0.5 * w, cy + 0.5 * h], axis=-1)
    return top_v, labels, boxes


@jax.jit
def kernel(cls0, reg0, cls1, reg1, cls2, reg2):
    per_level = [(cls0, reg0), (cls1, reg1), (cls2, reg2)]
    all_s, all_l, all_b = [], [], []
    for lvl, (c, r) in enumerate(per_level):
        s, l, b = _decode_level(c, r, _STRIDES[lvl], _ANCHOR_SIZE[lvl])
        all_s.append(s)
        all_l.append(l)
        all_b.append(b)
    scores = jnp.concatenate(all_s)
    labels = jnp.concatenate(all_l)
    boxes = jnp.concatenate(all_b)

    order = jnp.argsort(-scores)
    scores = scores[order]
    labels = labels[order]
    boxes = boxes[order]

    keep_f, sc_out, lb_out = _run_nms(boxes, scores, labels)
    return boxes, sc_out, lb_out, keep_f > 0.5
